# Initial kernel scaffold; baseline (speedup 1.0000x reference)
import jax
import jax.numpy as jnp
from jax.experimental import pallas as pl


def kernel(x, edge_index, batch, gc0_w, gc0_b, gc1_w, gc1_b, mu_w, mu_b, lv_w, lv_b, d0_w, d0_b, d1_w, d1_b, mx_w, mx_b, logvar_x_param):
    raise NotImplementedError("write your pallas kernel here")


# speedup vs baseline: 33.8747x; 33.8747x over previous
import functools

import jax
import jax.numpy as jnp
from jax import lax
from jax.experimental import pallas as pl
from jax.experimental.pallas import tpu as pltpu
from jax.experimental.pallas import tpu_sc as plsc

N = 10000
E = 320000
D = 128
H = 128
Z = 64
DH = 256
OUT = 231
G = 64

NP = 10240
EP = 327680
NC = 2
NS = 16
NW = NC * NS
CH = 128
CPW = EP // NW // CH
RPS = NP // NS
BR = 512
NBLK = NP // BR

_mesh = plsc.VectorSubcoreMesh(core_axis_name="c", subcore_axis_name="s")


def _zero_rows(buf, nrows):
    z = jnp.zeros((16,), jnp.float32)

    @pl.loop(0, nrows)
    def _(r):
        for k in range(8):
            buf[r, pl.ds(k * 16, 16)] = z


def _deg_body(dst_hbm, deg_out, dst_v, ones_v, zrow_v, deg_sp, semg):
    c = lax.axis_index("c")
    s = lax.axis_index("s")
    wid = s * NC + c

    @pl.loop(0, RPS // 16)
    def _(r):
        zrow_v[pl.ds(r * 16, 16)] = jnp.zeros((16,), jnp.float32)
    for k in range(8):
        ones_v[pl.ds(k * 16, 16)] = jnp.ones((16,), jnp.float32)
    pltpu.sync_copy(zrow_v, deg_sp.at[pl.ds(s * RPS, RPS)])
    plsc.subcore_barrier()

    pltpu.async_copy(dst_hbm.at[wid], dst_v, semg).wait()

    @pl.loop(0, CPW)
    def _(j):
        pltpu.sync_copy(ones_v, deg_sp.at[dst_v.at[j]], add=True)

    plsc.subcore_barrier()
    pltpu.sync_copy(deg_sp.at[pl.ds(s * RPS, RPS)], zrow_v)
    pltpu.sync_copy(zrow_v, deg_out.at[c, pl.ds(s * RPS, RPS)])


_deg_call = pl.kernel(
    _deg_body,
    out_type=jax.ShapeDtypeStruct((NC, NP), jnp.float32),
    mesh=_mesh,
    scratch_types=[
        pltpu.VMEM((CPW, CH), jnp.int32),
        pltpu.VMEM((CH,), jnp.float32),
        pltpu.VMEM((RPS,), jnp.float32),
        pltpu.VMEM_SHARED((NP,), jnp.float32),
        pltpu.SemaphoreType.DMA,
    ],
)


def _spmm_body(h_hbm, src_hbm, dst_hbm, out_hbm,
               sidx_a, sidx_b, dst_v, rows_a, rows_b, acc_sp,
               semg_a, semg_b, sems_a, sems_b):
    c = lax.axis_index("c")
    s = lax.axis_index("s")
    wid = s * NC + c

    _zero_rows(rows_a, CH)

    @pl.loop(0, RPS // CH)
    def _(r):
        pltpu.sync_copy(rows_a, acc_sp.at[pl.ds(s * RPS + r * CH, CH)])

    pltpu.async_copy(dst_hbm.at[wid], dst_v, semg_b).wait()
    plsc.subcore_barrier()

    pltpu.sync_copy(src_hbm.at[wid, 0], sidx_a)
    pltpu.async_copy(h_hbm.at[sidx_a], rows_a, semg_a)

    @pl.loop(0, CPW // 2)
    def _(jj):
        j = jj * 2
        pltpu.sync_copy(src_hbm.at[wid, j + 1], sidx_b)

        @pl.when(jj > 0)
        def _():
            pltpu.make_async_copy(rows_b, acc_sp.at[dst_v.at[j - 1]],
                                  sems_b).wait()
        pltpu.async_copy(h_hbm.at[sidx_b], rows_b, semg_b)
        pltpu.make_async_copy(h_hbm.at[sidx_a], rows_a, semg_a).wait()
        pltpu.async_copy(rows_a, acc_sp.at[dst_v.at[j]], sems_a, add=True)

        @pl.when(jj < CPW // 2 - 1)
        def _():
            pltpu.sync_copy(src_hbm.at[wid, j + 2], sidx_a)
            pltpu.make_async_copy(rows_a, acc_sp.at[dst_v.at[j]],
                                  sems_a).wait()
            pltpu.async_copy(h_hbm.at[sidx_a], rows_a, semg_a)

        @pl.when(jj == CPW // 2 - 1)
        def _():
            pltpu.make_async_copy(rows_a, acc_sp.at[dst_v.at[j]],
                                  sems_a).wait()
        pltpu.make_async_copy(h_hbm.at[sidx_b], rows_b, semg_b).wait()
        pltpu.async_copy(rows_b, acc_sp.at[dst_v.at[j + 1]], sems_b, add=True)

    pltpu.make_async_copy(rows_b, acc_sp.at[dst_v.at[CPW - 1]], sems_b).wait()
    plsc.subcore_barrier()

    @pl.loop(0, RPS // CH)
    def _(r):
        pltpu.sync_copy(acc_sp.at[pl.ds(s * RPS + r * CH, CH)], rows_a)
        pltpu.sync_copy(rows_a, out_hbm.at[c, pl.ds(s * RPS + r * CH, CH)])


def _make_spmm():
    return pl.kernel(
        _spmm_body,
        out_type=jax.ShapeDtypeStruct((NC, NP, H), jnp.float32),
        mesh=_mesh,
        scratch_types=[
            pltpu.VMEM((CH,), jnp.int32),
            pltpu.VMEM((CH,), jnp.int32),
            pltpu.VMEM((CPW, CH), jnp.int32),
            pltpu.VMEM((CH, H), jnp.float32),
            pltpu.VMEM((CH, H), jnp.float32),
            pltpu.VMEM_SHARED((NP, H), jnp.float32),
            pltpu.SemaphoreType.DMA,
            pltpu.SemaphoreType.DMA,
            pltpu.SemaphoreType.DMA,
            pltpu.SemaphoreType.DMA,
        ],
    )


_spmm_call = _make_spmm()


def _silu(v):
    return v / (1.0 + jnp.exp(-v))


def _dinv_of(deg_ref):
    return lax.rsqrt(deg_ref[0] + deg_ref[1] + 1.0)


def _tcA_body(deg_ref, x_ref, w_ref, o_ref):
    dinv = _dinv_of(deg_ref)
    h = jnp.dot(x_ref[...], w_ref[...], preferred_element_type=jnp.float32)
    o_ref[...] = h * dinv[:, None]


def _tcB_body(deg_ref, acc_ref, hp_ref, b_ref, w_ref, o_ref):
    dinv = _dinv_of(deg_ref)
    y = (acc_ref[0] + acc_ref[1] + hp_ref[...]) * dinv[:, None] + b_ref[...]
    h1 = _silu(y)
    o_ref[...] = jnp.dot(h1, w_ref[...],
                         preferred_element_type=jnp.float32) * dinv[:, None]


def _tcC_body(deg_ref, acc_ref, hp_ref, b_ref, o_ref):
    dinv = _dinv_of(deg_ref)
    y = (acc_ref[0] + acc_ref[1] + hp_ref[...]) * dinv[:, None] + b_ref[...]
    o_ref[...] = _silu(y) * dinv[:, None]


def _tcD_body(deg_ref, acc_ref, hp_ref, bt_ref,
              muw_ref, mub_ref, lvw_ref, lvb_ref, eps_ref,
              d0w_ref, d0b_ref, d1w_ref, d1b_ref, mxw_ref, mxb_ref, lx_ref,
              omu_ref, olv_ref, omx_ref, olx_ref, qp_ref, cnt_ref):
    i = pl.program_id(0)

    @pl.when(i == 0)
    def _():
        qp_ref[...] = jnp.zeros_like(qp_ref)
        cnt_ref[...] = jnp.zeros_like(cnt_ref)

    dinv = _dinv_of(deg_ref)
    y3 = (acc_ref[0] + acc_ref[1] + hp_ref[...]) * dinv[:, None]
    gids = lax.broadcasted_iota(jnp.int32, (G, BR), 0)
    msk = (bt_ref[...] == gids).astype(jnp.float32)
    qp_ref[...] += jnp.dot(msk, y3, preferred_element_type=jnp.float32)
    cnt_ref[...] += jnp.sum(msk, axis=1)[None, :]

    @pl.when(i == NBLK - 1)
    def _():
        qp = qp_ref[...] / jnp.maximum(cnt_ref[0], 1.0)[:, None]
        mu = jnp.dot(qp, muw_ref[...],
                     preferred_element_type=jnp.float32) + mub_ref[...]
        lv = jnp.dot(qp, lvw_ref[...],
                     preferred_element_type=jnp.float32) + lvb_ref[...]
        z = mu + jnp.exp(0.5 * lv) * eps_ref[...]
        hd = jnp.tanh(jnp.dot(z, d0w_ref[...],
                              preferred_element_type=jnp.float32) + d0b_ref[...])
        hd = jnp.tanh(jnp.dot(hd, d1w_ref[...],
                              preferred_element_type=jnp.float32) + d1b_ref[...])
        mx = jnp.dot(hd, mxw_ref[...],
                     preferred_element_type=jnp.float32) + mxb_ref[...]
        omu_ref[...] = mu
        olv_ref[...] = lv
        omx_ref[...] = mx
        olx_ref[...] = jnp.broadcast_to(lx_ref[...], (G, OUT))


def _whole(shape):
    nd = len(shape)
    return pl.BlockSpec(shape, lambda i: (0,) * nd)


_deg_spec = pl.BlockSpec((2, BR), lambda i: (0, i))
_row_spec = pl.BlockSpec((BR, H), lambda i: (i, 0))
_acc_spec = pl.BlockSpec((2, BR, H), lambda i: (0, i, 0))

_tcA_call = pl.pallas_call(
    _tcA_body,
    grid=(NBLK,),
    in_specs=[_deg_spec, _row_spec, _whole((D, H))],
    out_specs=_row_spec,
    out_shape=jax.ShapeDtypeStruct((NP, H), jnp.float32),
)

_tcB_call = pl.pallas_call(
    _tcB_body,
    grid=(NBLK,),
    in_specs=[_deg_spec, _acc_spec, _row_spec, _whole((1, H)), _whole((H, H))],
    out_specs=_row_spec,
    out_shape=jax.ShapeDtypeStruct((NP, H), jnp.float32),
)

_tcC_call = pl.pallas_call(
    _tcC_body,
    grid=(NBLK,),
    in_specs=[_deg_spec, _acc_spec, _row_spec, _whole((1, H))],
    out_specs=_row_spec,
    out_shape=jax.ShapeDtypeStruct((NP, H), jnp.float32),
)

_tcD_call = pl.pallas_call(
    _tcD_body,
    grid=(NBLK,),
    in_specs=[_deg_spec, _acc_spec, _row_spec,
              pl.BlockSpec((1, BR), lambda i: (0, i)),
              _whole((H, Z)), _whole((1, Z)), _whole((H, Z)), _whole((1, Z)),
              _whole((G, Z)),
              _whole((Z, DH)), _whole((1, DH)), _whole((DH, DH)),
              _whole((1, DH)), _whole((DH, OUT)), _whole((1, OUT)),
              _whole((1, OUT))],
    out_specs=[_whole((G, Z)), _whole((G, Z)), _whole((G, OUT)),
               _whole((G, OUT))],
    out_shape=[jax.ShapeDtypeStruct((G, Z), jnp.float32),
               jax.ShapeDtypeStruct((G, Z), jnp.float32),
               jax.ShapeDtypeStruct((G, OUT), jnp.float32),
               jax.ShapeDtypeStruct((G, OUT), jnp.float32)],
    scratch_shapes=[pltpu.VMEM((G, H), jnp.float32),
                    pltpu.VMEM((1, G), jnp.float32)],
)


def kernel(x, edge_index, batch, gc0_w, gc0_b, gc1_w, gc1_b, mu_w, mu_b,
           lv_w, lv_b, d0_w, d0_b, d1_w, d1_b, mx_w, mx_b, logvar_x_param):
    pad = N + (jnp.arange(EP - E, dtype=jnp.int32) % (NP - N))
    srcp = jnp.concatenate([edge_index[0], pad]).reshape(NW, CPW, CH)
    dstp = jnp.concatenate([edge_index[1], pad]).reshape(NW, CPW, CH)
    xp = jnp.pad(x, ((0, NP - N), (0, 0)))
    bt = jnp.pad(batch, (0, NP - N), constant_values=G).reshape(1, NP)
    eps = jax.random.normal(jax.random.key(42), (G, Z), jnp.float32)
    b0 = gc0_b.reshape(1, H)
    b1 = gc1_b.reshape(1, H)

    degs = _deg_call(dstp)
    h0p = _tcA_call(degs, xp, gc0_w)
    acc1 = _spmm_call(h0p, srcp, dstp)
    h1p = _tcB_call(degs, acc1, h0p, b0, gc1_w)
    acc2 = _spmm_call(h1p, srcp, dstp)
    h2p = _tcC_call(degs, acc2, h1p, b1)
    acc3 = _spmm_call(h2p, srcp, dstp)
    mu_zp, logvar_zp, mu_x, logvar_x = _tcD_call(
        degs, acc3, h2p, bt,
        mu_w, mu_b.reshape(1, Z), lv_w, lv_b.reshape(1, Z), eps,
        d0_w, d0_b.reshape(1, DH), d1_w, d1_b.reshape(1, DH),
        mx_w, mx_b.reshape(1, OUT), logvar_x_param.reshape(1, OUT))
    return (mu_zp, logvar_zp, mu_x, logvar_x)
